# Initial kernel scaffold; baseline (speedup 1.0000x reference)
import functools
import jax
import jax.numpy as jnp
from jax import lax
from jax.experimental import pallas as pl
from jax.experimental.pallas import tpu as pltpu, tpu_sc as plsc

NN = 10000
HH = 128
GG = 64
CC = 10
NC = 2
NS = 16
NW = NC * NS
EB = 128
NPAD = 10240
RPT = NPAD // NS
RTC = 640
NB = NPAD // RTC

_mesh = plsc.VectorSubcoreMesh(
    core_axis_name="c", subcore_axis_name="s", num_cores=NC, num_subcores=NS)


def _sc_deg_body(nper, dst_hbm, ones_hbm, zeros_hbm, out_hbm,
                 dstbuf, onesbuf, stage, acc, sem):
  c = lax.axis_index("c")
  s = lax.axis_index("s")
  wid = s * NC + c
  r0 = s * RPT
  pltpu.sync_copy(zeros_hbm.at[pl.ds(0, RPT)], stage)
  pltpu.sync_copy(stage, acc.at[pl.ds(r0, RPT)])
  pltpu.sync_copy(ones_hbm, onesbuf)
  plsc.subcore_barrier()

  def chunk(i, carry):
    base = (wid * nper + i) * EB
    pltpu.sync_copy(dst_hbm.at[pl.ds(base, EB)], dstbuf)
    pltpu.sync_copy(onesbuf, acc.at[dstbuf], add=True)
    return carry

  lax.fori_loop(0, nper, chunk, 0)
  plsc.subcore_barrier()
  pltpu.sync_copy(acc.at[pl.ds(r0, RPT)], stage)
  pltpu.sync_copy(stage, out_hbm.at[c, pl.ds(r0, RPT)])


def _sc_deg(dst_p, ones16, zeros16, nper):
  return pl.kernel(
      functools.partial(_sc_deg_body, nper),
      out_type=jax.ShapeDtypeStruct((NC, NPAD, 16), jnp.float32),
      mesh=_mesh,
      scratch_types=[
          pltpu.VMEM((EB,), jnp.int32),
          pltpu.VMEM((EB, 16), jnp.float32),
          pltpu.VMEM((RPT, 16), jnp.float32),
          pltpu.VMEM_SHARED((NPAD, 16), jnp.float32),
          pltpu.SemaphoreType.DMA,
      ],
  )(dst_p, ones16, zeros16)


def _sc_agg_body(nper, g_hbm, src_hbm, dst_hbm, zeros_hbm, out_hbm,
                 srcbuf, dstbuf, rowbuf, stage, acc, sem):
  c = lax.axis_index("c")
  s = lax.axis_index("s")
  wid = s * NC + c
  r0 = s * RPT
  for q in range(RPT // EB):
    pltpu.sync_copy(zeros_hbm.at[pl.ds(q * EB, EB)], stage)
    pltpu.sync_copy(stage, acc.at[pl.ds(r0 + q * EB, EB)])
  plsc.subcore_barrier()

  def chunk(i, carry):
    base = (wid * nper + i) * EB
    pltpu.sync_copy(src_hbm.at[pl.ds(base, EB)], srcbuf)
    pltpu.sync_copy(dst_hbm.at[pl.ds(base, EB)], dstbuf)
    pltpu.async_copy(g_hbm.at[srcbuf], rowbuf, sem).wait()
    pltpu.sync_copy(rowbuf, acc.at[dstbuf], add=True)
    return carry

  lax.fori_loop(0, nper, chunk, 0)
  plsc.subcore_barrier()
  for q in range(RPT // EB):
    pltpu.sync_copy(acc.at[pl.ds(r0 + q * EB, EB)], stage)
    pltpu.sync_copy(stage, out_hbm.at[c, pl.ds(r0 + q * EB, EB)])


def _sc_agg(g, src_p, dst_p, zeros128, nper):
  return pl.kernel(
      functools.partial(_sc_agg_body, nper),
      out_type=jax.ShapeDtypeStruct((NC, NPAD, HH), jnp.float32),
      mesh=_mesh,
      scratch_types=[
          pltpu.VMEM((EB,), jnp.int32),
          pltpu.VMEM((EB,), jnp.int32),
          pltpu.VMEM((EB, HH), jnp.float32),
          pltpu.VMEM((EB, HH), jnp.float32),
          pltpu.VMEM_SHARED((NPAD, HH), jnp.float32),
          pltpu.SemaphoreType.DMA,
      ],
  )(g, src_p, dst_p, zeros128)


def _dis(deg_ref):
  d = deg_ref[...]
  deg = d[0, :, 0] + d[1, :, 0] + 1.0
  return lax.rsqrt(deg)


def _tc_g0_body(x_ref, w_ref, deg_ref, o_ref):
  dis = _dis(deg_ref)
  o_ref[...] = jnp.dot(x_ref[...], w_ref[...],
                       preferred_element_type=jnp.float32) * dis[:, None]


def _tc_g0(x_pad, W0, degp):
  return pl.pallas_call(
      _tc_g0_body,
      grid=(NB,),
      in_specs=[
          pl.BlockSpec((RTC, HH), lambda i: (i, 0)),
          pl.BlockSpec((HH, HH), lambda i: (0, 0)),
          pl.BlockSpec((NC, RTC, 16), lambda i: (0, i, 0)),
      ],
      out_specs=pl.BlockSpec((RTC, HH), lambda i: (i, 0)),
      out_shape=jax.ShapeDtypeStruct((NPAD, HH), jnp.float32),
  )(x_pad, W0, degp)


def _tc_layer_body(p_ref, g_ref, deg_ref, b_ref, w_ref, o_ref):
  dis = _dis(deg_ref)
  pv = p_ref[...]
  acc = pv[0] + pv[1] + g_ref[...]
  t = jnp.maximum(acc * dis[:, None] + b_ref[...], 0.0)
  o_ref[...] = jnp.dot(t, w_ref[...],
                       preferred_element_type=jnp.float32) * dis[:, None]


def _tc_layer(p, g, degp, b, Wn):
  return pl.pallas_call(
      _tc_layer_body,
      grid=(NB,),
      in_specs=[
          pl.BlockSpec((NC, RTC, HH), lambda i: (0, i, 0)),
          pl.BlockSpec((RTC, HH), lambda i: (i, 0)),
          pl.BlockSpec((NC, RTC, 16), lambda i: (0, i, 0)),
          pl.BlockSpec((1, HH), lambda i: (0, 0)),
          pl.BlockSpec((HH, HH), lambda i: (0, 0)),
      ],
      out_specs=pl.BlockSpec((RTC, HH), lambda i: (i, 0)),
      out_shape=jax.ShapeDtypeStruct((NPAD, HH), jnp.float32),
  )(p, g, degp, b, Wn)


def _tc_final_body(p_ref, g_ref, deg_ref, b_ref, batch_ref, fcw_ref, fcb_ref,
                   f2w_ref, f2b_ref, o_ref, pooled, cnt):
  i = pl.program_id(0)

  @pl.when(i == 0)
  def _():
    pooled[...] = jnp.zeros_like(pooled)
    cnt[...] = jnp.zeros_like(cnt)

  dis = _dis(deg_ref)
  pv = p_ref[...]
  acc = pv[0] + pv[1] + g_ref[...]
  h = jnp.maximum(acc * dis[:, None] + b_ref[...], 0.0)
  bvec = batch_ref[...][0, 0, :]
  onehot = (bvec[:, None] == lax.broadcasted_iota(jnp.int32, (RTC, GG), 1)
            ).astype(jnp.float32)
  pooled[...] += lax.dot_general(onehot, h, (((0,), (0,)), ((), ())),
                                 preferred_element_type=jnp.float32)
  cnt[...] += lax.dot_general(onehot, jnp.ones((RTC, HH), jnp.float32),
                              (((0,), (0,)), ((), ())),
                              preferred_element_type=jnp.float32)

  @pl.when(i == NB - 1)
  def _():
    pm = pooled[...] / jnp.maximum(cnt[...], 1.0)
    z = jnp.maximum(jnp.dot(pm, fcw_ref[...],
                            preferred_element_type=jnp.float32)
                    + fcb_ref[...], 0.0)
    o_ref[...] = jnp.dot(z, f2w_ref[...],
                         preferred_element_type=jnp.float32) + f2b_ref[...]


def _tc_final(p, g, degp, b, batch3, fcW, fcb, f2w, f2b):
  return pl.pallas_call(
      _tc_final_body,
      grid=(NB,),
      in_specs=[
          pl.BlockSpec((NC, RTC, HH), lambda i: (0, i, 0)),
          pl.BlockSpec((RTC, HH), lambda i: (i, 0)),
          pl.BlockSpec((NC, RTC, 16), lambda i: (0, i, 0)),
          pl.BlockSpec((1, HH), lambda i: (0, 0)),
          pl.BlockSpec((1, 1, RTC), lambda i: (i, 0, 0)),
          pl.BlockSpec((HH, HH), lambda i: (0, 0)),
          pl.BlockSpec((1, HH), lambda i: (0, 0)),
          pl.BlockSpec((HH, HH), lambda i: (0, 0)),
          pl.BlockSpec((1, HH), lambda i: (0, 0)),
      ],
      out_specs=pl.BlockSpec((GG, HH), lambda i: (0, 0)),
      out_shape=jax.ShapeDtypeStruct((GG, HH), jnp.float32),
      scratch_shapes=[
          pltpu.VMEM((GG, HH), jnp.float32),
          pltpu.VMEM((GG, HH), jnp.float32),
      ],
  )(p, g, degp, b, batch3, fcW, fcb, f2w, f2b)


def kernel(x, edge_index, batch, W0, b0, W1, b1, W2, b2, fcW, fcb, fc2W, fc2b):
  E = edge_index.shape[1]
  EP = -(-E // (NW * EB)) * (NW * EB)
  nper = EP // (NW * EB)
  src = edge_index[0]
  dst = edge_index[1]
  npadrows = NPAD - NN
  pad_idx = (NN + (jnp.arange(EP - E, dtype=jnp.int32) % npadrows)
             ).astype(jnp.int32)
  src_p = jnp.concatenate([src, pad_idx])
  dst_p = jnp.concatenate([dst, pad_idx])
  x_pad = jnp.pad(x, ((0, npadrows), (0, 0)))
  zeros128 = jnp.zeros((NPAD, HH), jnp.float32)
  zeros16 = jnp.zeros((NPAD, 16), jnp.float32)
  ones16 = jnp.ones((EB, 16), jnp.float32)

  degp = _sc_deg(dst_p, ones16, zeros16, nper)
  g0 = _tc_g0(x_pad, W0, degp)
  p = _sc_agg(g0, src_p, dst_p, zeros128, nper)
  g1 = _tc_layer(p, g0, degp, b0.reshape(1, HH), W1)
  p = _sc_agg(g1, src_p, dst_p, zeros128, nper)
  g2 = _tc_layer(p, g1, degp, b1.reshape(1, HH), W2)
  p = _sc_agg(g2, src_p, dst_p, zeros128, nper)

  batch3 = jnp.concatenate(
      [batch.astype(jnp.int32), jnp.full((npadrows,), GG, jnp.int32)]
  ).reshape(NB, 1, RTC)
  f2w = jnp.pad(fc2W, ((0, 0), (0, HH - CC)))
  f2b = jnp.pad(fc2b, (0, HH - CC)).reshape(1, HH)
  out128 = _tc_final(p, g2, degp, b2.reshape(1, HH), batch3,
                     fcW, fcb.reshape(1, HH), f2w, f2b)
  return out128[:, :CC]


# speedup vs baseline: 11.8241x; 11.8241x over previous
import functools
import jax
import jax.numpy as jnp
from jax import lax
from jax.experimental import pallas as pl
from jax.experimental.pallas import tpu as pltpu, tpu_sc as plsc

NN = 10000
HH = 128
GG = 64
CC = 10
NC = 2
NS = 16
NW = NC * NS
EB = 128
NPAD = 10240
RPT = NPAD // NS
RTC = 640
NB = NPAD // RTC

@functools.lru_cache(maxsize=1)
def _mesh():
  return plsc.VectorSubcoreMesh(
      core_axis_name="c", subcore_axis_name="s", num_cores=NC, num_subcores=NS)


JJ = EB // 16
NQ = RPT // 16


def _sc_deg_body(nper, dst_hbm, ones_hbm, zeros_hbm, iota_hbm, out_hbm,
                 idxd, onesbuf, zbuf, iotabuf, stage, acc, sem):
  c = lax.axis_index("c")
  s = lax.axis_index("s")
  wid = s * NC + c
  q0 = s * NQ
  pltpu.sync_copy(ones_hbm, onesbuf)
  pltpu.sync_copy(zeros_hbm, zbuf)
  pltpu.sync_copy(iota_hbm.at[pl.ds(q0, NQ)], iotabuf)

  def initq(q, carry):
    pltpu.sync_copy(zbuf, acc.at[iotabuf.at[q]])
    return carry

  lax.fori_loop(0, NQ, initq, 0)
  plsc.subcore_barrier()

  def chunk(i, carry):
    base = (wid * nper + i) * JJ
    pltpu.sync_copy(dst_hbm.at[pl.ds(base, JJ)], idxd)
    descs = [pltpu.async_copy(onesbuf, acc.at[idxd.at[j]], sem, add=True)
             for j in range(JJ)]
    for d in descs:
      d.wait()
    return carry

  lax.fori_loop(0, nper, chunk, 0)
  plsc.subcore_barrier()

  def dumpq(q, carry):
    pltpu.async_copy(acc.at[iotabuf.at[q]], stage, sem).wait()
    pltpu.sync_copy(stage, out_hbm.at[c, pl.ds((q0 + q) * 16, 16)])
    return carry

  lax.fori_loop(0, NQ, dumpq, 0)


def _sc_deg(dst2, ones128, zeros128, iota2, nper):
  return pl.kernel(
      functools.partial(_sc_deg_body, nper),
      out_type=jax.ShapeDtypeStruct((NC, NPAD, HH), jnp.float32),
      mesh=_mesh(),
      scratch_types=[
          pltpu.VMEM((JJ, 16), jnp.int32),
          pltpu.VMEM((16, HH), jnp.float32),
          pltpu.VMEM((16, HH), jnp.float32),
          pltpu.VMEM((NQ, 16), jnp.int32),
          pltpu.VMEM((16, HH), jnp.float32),
          pltpu.VMEM_SHARED((NPAD, HH), jnp.float32),
          pltpu.SemaphoreType.DMA,
      ],
  )(dst2, ones128, zeros128, iota2)


def _sc_agg_body(nper, g_hbm, src_hbm, dst_hbm, zeros_hbm, iota_hbm, out_hbm,
                 idxs, idxd, zbuf, iotabuf, rowbuf, stage, acc, sem):
  c = lax.axis_index("c")
  s = lax.axis_index("s")
  wid = s * NC + c
  q0 = s * NQ
  pltpu.sync_copy(zeros_hbm, zbuf)
  pltpu.sync_copy(iota_hbm.at[pl.ds(q0, NQ)], iotabuf)

  def initq(q, carry):
    pltpu.sync_copy(zbuf, acc.at[iotabuf.at[q]])
    return carry

  lax.fori_loop(0, NQ, initq, 0)
  plsc.subcore_barrier()

  def chunk(i, carry):
    base = (wid * nper + i) * JJ
    pltpu.sync_copy(src_hbm.at[pl.ds(base, JJ)], idxs)
    pltpu.sync_copy(dst_hbm.at[pl.ds(base, JJ)], idxd)
    descs = [pltpu.async_copy(g_hbm.at[idxs.at[j]],
                              rowbuf.at[pl.ds(16 * j, 16)], sem)
             for j in range(JJ)]
    for d in descs:
      d.wait()
    descs2 = [pltpu.async_copy(rowbuf.at[pl.ds(16 * j, 16)],
                               acc.at[idxd.at[j]], sem, add=True)
              for j in range(JJ)]
    for d in descs2:
      d.wait()
    return carry

  lax.fori_loop(0, nper, chunk, 0)
  plsc.subcore_barrier()

  def dumpq(q, carry):
    pltpu.async_copy(acc.at[iotabuf.at[q]], stage, sem).wait()
    pltpu.sync_copy(stage, out_hbm.at[c, pl.ds((q0 + q) * 16, 16)])
    return carry

  lax.fori_loop(0, NQ, dumpq, 0)


def _sc_agg(g, src2, dst2, zeros128, iota2, nper):
  return pl.kernel(
      functools.partial(_sc_agg_body, nper),
      out_type=jax.ShapeDtypeStruct((NC, NPAD, HH), jnp.float32),
      mesh=_mesh(),
      scratch_types=[
          pltpu.VMEM((JJ, 16), jnp.int32),
          pltpu.VMEM((JJ, 16), jnp.int32),
          pltpu.VMEM((16, HH), jnp.float32),
          pltpu.VMEM((NQ, 16), jnp.int32),
          pltpu.VMEM((EB, HH), jnp.float32),
          pltpu.VMEM((16, HH), jnp.float32),
          pltpu.VMEM_SHARED((NPAD, HH), jnp.float32),
          pltpu.SemaphoreType.DMA,
      ],
  )(g, src2, dst2, zeros128, iota2)


def _dis(deg_ref):
  d = deg_ref[...]
  deg = d[0, :, 0] + d[1, :, 0] + 1.0
  return lax.rsqrt(deg)


def _tc_g0_body(x_ref, w_ref, deg_ref, o_ref):
  dis = _dis(deg_ref)
  o_ref[...] = jnp.dot(x_ref[...], w_ref[...],
                       preferred_element_type=jnp.float32) * dis[:, None]


def _tc_g0(x_pad, W0, degp):
  return pl.pallas_call(
      _tc_g0_body,
      grid=(NB,),
      in_specs=[
          pl.BlockSpec((RTC, HH), lambda i: (i, 0)),
          pl.BlockSpec((HH, HH), lambda i: (0, 0)),
          pl.BlockSpec((NC, RTC, HH), lambda i: (0, i, 0)),
      ],
      out_specs=pl.BlockSpec((RTC, HH), lambda i: (i, 0)),
      out_shape=jax.ShapeDtypeStruct((NPAD, HH), jnp.float32),
  )(x_pad, W0, degp)


def _tc_layer_body(p_ref, g_ref, deg_ref, b_ref, w_ref, o_ref):
  dis = _dis(deg_ref)
  pv = p_ref[...]
  acc = pv[0] + pv[1] + g_ref[...]
  t = jnp.maximum(acc * dis[:, None] + b_ref[...], 0.0)
  o_ref[...] = jnp.dot(t, w_ref[...],
                       preferred_element_type=jnp.float32) * dis[:, None]


def _tc_layer(p, g, degp, b, Wn):
  return pl.pallas_call(
      _tc_layer_body,
      grid=(NB,),
      in_specs=[
          pl.BlockSpec((NC, RTC, HH), lambda i: (0, i, 0)),
          pl.BlockSpec((RTC, HH), lambda i: (i, 0)),
          pl.BlockSpec((NC, RTC, HH), lambda i: (0, i, 0)),
          pl.BlockSpec((1, HH), lambda i: (0, 0)),
          pl.BlockSpec((HH, HH), lambda i: (0, 0)),
      ],
      out_specs=pl.BlockSpec((RTC, HH), lambda i: (i, 0)),
      out_shape=jax.ShapeDtypeStruct((NPAD, HH), jnp.float32),
  )(p, g, degp, b, Wn)


def _tc_final_body(p_ref, g_ref, deg_ref, b_ref, batch_ref, fcw_ref, fcb_ref,
                   f2w_ref, f2b_ref, o_ref, pooled, cnt):
  i = pl.program_id(0)

  @pl.when(i == 0)
  def _():
    pooled[...] = jnp.zeros_like(pooled)
    cnt[...] = jnp.zeros_like(cnt)

  dis = _dis(deg_ref)
  pv = p_ref[...]
  acc = pv[0] + pv[1] + g_ref[...]
  h = jnp.maximum(acc * dis[:, None] + b_ref[...], 0.0)
  bvec = batch_ref[...][0, 0, :]
  onehot = (bvec[:, None] == lax.broadcasted_iota(jnp.int32, (RTC, GG), 1)
            ).astype(jnp.float32)
  pooled[...] += lax.dot_general(onehot, h, (((0,), (0,)), ((), ())),
                                 preferred_element_type=jnp.float32)
  cnt[...] += lax.dot_general(onehot, jnp.ones((RTC, HH), jnp.float32),
                              (((0,), (0,)), ((), ())),
                              preferred_element_type=jnp.float32)

  @pl.when(i == NB - 1)
  def _():
    pm = pooled[...] / jnp.maximum(cnt[...], 1.0)
    z = jnp.maximum(jnp.dot(pm, fcw_ref[...],
                            preferred_element_type=jnp.float32)
                    + fcb_ref[...], 0.0)
    o_ref[...] = jnp.dot(z, f2w_ref[...],
                         preferred_element_type=jnp.float32) + f2b_ref[...]


def _tc_final(p, g, degp, b, batch3, fcW, fcb, f2w, f2b):
  return pl.pallas_call(
      _tc_final_body,
      grid=(NB,),
      in_specs=[
          pl.BlockSpec((NC, RTC, HH), lambda i: (0, i, 0)),
          pl.BlockSpec((RTC, HH), lambda i: (i, 0)),
          pl.BlockSpec((NC, RTC, HH), lambda i: (0, i, 0)),
          pl.BlockSpec((1, HH), lambda i: (0, 0)),
          pl.BlockSpec((1, 1, RTC), lambda i: (i, 0, 0)),
          pl.BlockSpec((HH, HH), lambda i: (0, 0)),
          pl.BlockSpec((1, HH), lambda i: (0, 0)),
          pl.BlockSpec((HH, HH), lambda i: (0, 0)),
          pl.BlockSpec((1, HH), lambda i: (0, 0)),
      ],
      out_specs=pl.BlockSpec((GG, HH), lambda i: (0, 0)),
      out_shape=jax.ShapeDtypeStruct((GG, HH), jnp.float32),
      scratch_shapes=[
          pltpu.VMEM((GG, HH), jnp.float32),
          pltpu.VMEM((GG, HH), jnp.float32),
      ],
  )(p, g, degp, b, batch3, fcW, fcb, f2w, f2b)


def kernel(x, edge_index, batch, W0, b0, W1, b1, W2, b2, fcW, fcb, fc2W, fc2b):
  E = edge_index.shape[1]
  EP = -(-E // (NW * EB)) * (NW * EB)
  nper = EP // (NW * EB)
  src = edge_index[0]
  dst = edge_index[1]
  npadrows = NPAD - NN
  pad_idx = (NN + (jnp.arange(EP - E, dtype=jnp.int32) % npadrows)
             ).astype(jnp.int32)
  src2 = jnp.concatenate([src, pad_idx]).reshape(EP // 16, 16)
  dst2 = jnp.concatenate([dst, pad_idx]).reshape(EP // 16, 16)
  x_pad = jnp.pad(x, ((0, npadrows), (0, 0)))
  zeros128 = jnp.zeros((16, HH), jnp.float32)
  ones128 = jnp.ones((16, HH), jnp.float32)
  iota2 = jnp.arange(NPAD, dtype=jnp.int32).reshape(NPAD // 16, 16)

  degp = _sc_deg(dst2, ones128, zeros128, iota2, nper)
  g0 = _tc_g0(x_pad, W0, degp)
  p = _sc_agg(g0, src2, dst2, zeros128, iota2, nper)
  g1 = _tc_layer(p, g0, degp, b0.reshape(1, HH), W1)
  p = _sc_agg(g1, src2, dst2, zeros128, iota2, nper)
  g2 = _tc_layer(p, g1, degp, b1.reshape(1, HH), W2)
  p = _sc_agg(g2, src2, dst2, zeros128, iota2, nper)

  batch3 = jnp.concatenate(
      [batch.astype(jnp.int32), jnp.full((npadrows,), GG, jnp.int32)]
  ).reshape(NB, 1, RTC)
  f2w = jnp.pad(fc2W, ((0, 0), (0, HH - CC)))
  f2b = jnp.pad(fc2b, (0, HH - CC)).reshape(1, HH)
  out128 = _tc_final(p, g2, degp, b2.reshape(1, HH), batch3,
                     fcW, fcb.reshape(1, HH), f2w, f2b)
  return out128[:, :CC]


# speedup vs baseline: 21.7480x; 1.8393x over previous
import functools
import jax
import jax.numpy as jnp
from jax import lax
from jax.experimental import pallas as pl
from jax.experimental.pallas import tpu as pltpu, tpu_sc as plsc

NN = 10000
HH = 128
GG = 64
CC = 10
NC = 2
NS = 16
NW = NC * NS
EB = 128
NPAD = 10240
RPT = NPAD // NS
RTC = 640
NB = NPAD // RTC

@functools.lru_cache(maxsize=1)
def _mesh():
  return plsc.VectorSubcoreMesh(
      core_axis_name="c", subcore_axis_name="s", num_cores=NC, num_subcores=NS)


JJ = EB // 16
NQ = RPT // 16


def _sc_deg_body(nper, dst_hbm, ones_hbm, zeros_hbm, iota_hbm, out_hbm,
                 idxd, onesbuf, zbuf, iotabuf, stage, acc, sem, semi):
  c = lax.axis_index("c")
  s = lax.axis_index("s")
  wid = s * NC + c
  q0 = s * NQ
  pltpu.sync_copy(ones_hbm, onesbuf)
  pltpu.sync_copy(zeros_hbm, zbuf)
  pltpu.sync_copy(iota_hbm.at[pl.ds(q0, NQ)], iotabuf)

  def initq(q, carry):
    pltpu.sync_copy(zbuf, acc.at[iotabuf.at[q]])
    return carry

  lax.fori_loop(0, NQ, initq, 0)
  plsc.subcore_barrier()

  ebase = wid * nper * JJ
  pltpu.async_copy(dst_hbm.at[pl.ds(ebase, JJ)], idxd.at[pl.ds(0, JJ)], semi)

  def chunk(i, carry):
    k0 = lax.rem(i, 3) * JJ
    k1 = lax.rem(i + 1, 3) * JJ

    @pl.when(i < nper)
    def _():
      pltpu.make_async_copy(dst_hbm.at[pl.ds(ebase, JJ)],
                            idxd.at[pl.ds(k0, JJ)], semi).wait()
      for j in range(JJ):
        pltpu.async_copy(onesbuf, acc.at[idxd.at[k0 + j]], sem, add=True)

    @pl.when(i + 1 < nper)
    def _():
      pltpu.async_copy(dst_hbm.at[pl.ds(ebase + (i + 1) * JJ, JJ)],
                       idxd.at[pl.ds(k1, JJ)], semi)

    @pl.when(i >= 1)
    def _():
      for j in range(JJ):
        pltpu.make_async_copy(onesbuf, acc.at[iotabuf.at[0]], sem).wait()

    return carry

  lax.fori_loop(0, nper + 1, chunk, 0)
  plsc.subcore_barrier()

  def dumpq(q, carry):
    pltpu.async_copy(acc.at[iotabuf.at[q]], stage, sem).wait()
    pltpu.sync_copy(stage, out_hbm.at[c, pl.ds((q0 + q) * 16, 16)])
    return carry

  lax.fori_loop(0, NQ, dumpq, 0)


def _sc_deg(dst2, ones128, zeros128, iota2, nper):
  return pl.kernel(
      functools.partial(_sc_deg_body, nper),
      out_type=jax.ShapeDtypeStruct((NC, NPAD, HH), jnp.float32),
      mesh=_mesh(),
      scratch_types=[
          pltpu.VMEM((3 * JJ, 16), jnp.int32),
          pltpu.VMEM((16, HH), jnp.float32),
          pltpu.VMEM((16, HH), jnp.float32),
          pltpu.VMEM((NQ, 16), jnp.int32),
          pltpu.VMEM((16, HH), jnp.float32),
          pltpu.VMEM_SHARED((NPAD, HH), jnp.float32),
          pltpu.SemaphoreType.DMA,
          pltpu.SemaphoreType.DMA,
      ],
  )(dst2, ones128, zeros128, iota2)


def _sc_agg_body(nper, g_hbm, src_hbm, dst_hbm, zeros_hbm, iota_hbm, out_hbm,
                 idxs3, idxd3, zbuf, iotabuf, rowbuf2, stage, acc,
                 semg, sems, semi):
  c = lax.axis_index("c")
  s = lax.axis_index("s")
  wid = s * NC + c
  q0 = s * NQ
  pltpu.sync_copy(zeros_hbm, zbuf)
  pltpu.sync_copy(iota_hbm.at[pl.ds(q0, NQ)], iotabuf)
  ebase = wid * nper * JJ

  def initq(q, carry):
    pltpu.sync_copy(zbuf, acc.at[iotabuf.at[q]])
    return carry

  lax.fori_loop(0, NQ, initq, 0)
  pltpu.async_copy(src_hbm.at[pl.ds(ebase, JJ)], idxs3.at[pl.ds(0, JJ)], semi)
  pltpu.async_copy(dst_hbm.at[pl.ds(ebase, JJ)], idxd3.at[pl.ds(0, JJ)], semi)
  plsc.subcore_barrier()

  def step(i, carry):
    par = lax.rem(i, 2) * EB
    ppar = lax.rem(i + 1, 2) * EB
    k0 = lax.rem(i, 3) * JJ
    k1 = lax.rem(i + 1, 3) * JJ
    k2 = lax.rem(i + 2, 3) * JJ

    @pl.when(i >= 2)
    def _():
      for j in range(JJ):
        pltpu.make_async_copy(rowbuf2.at[pl.ds(par + 16 * j, 16)],
                              acc.at[iotabuf.at[0]], sems).wait()

    @pl.when(i < nper)
    def _():
      pltpu.make_async_copy(src_hbm.at[pl.ds(ebase, JJ)],
                            idxs3.at[pl.ds(k0, JJ)], semi).wait()
      pltpu.make_async_copy(dst_hbm.at[pl.ds(ebase, JJ)],
                            idxd3.at[pl.ds(k0, JJ)], semi).wait()
      for j in range(JJ):
        pltpu.async_copy(g_hbm.at[idxs3.at[k0 + j]],
                         rowbuf2.at[pl.ds(par + 16 * j, 16)], semg)

    @pl.when(jnp.logical_and(i >= 1, i <= nper))
    def _():
      for j in range(JJ):
        pltpu.make_async_copy(g_hbm.at[iotabuf.at[0]],
                              rowbuf2.at[pl.ds(ppar + 16 * j, 16)],
                              semg).wait()
      for j in range(JJ):
        pltpu.async_copy(rowbuf2.at[pl.ds(ppar + 16 * j, 16)],
                         acc.at[idxd3.at[k2 + j]], sems, add=True)

    @pl.when(i + 1 < nper)
    def _():
      nbase = ebase + (i + 1) * JJ
      pltpu.async_copy(src_hbm.at[pl.ds(nbase, JJ)],
                       idxs3.at[pl.ds(k1, JJ)], semi)
      pltpu.async_copy(dst_hbm.at[pl.ds(nbase, JJ)],
                       idxd3.at[pl.ds(k1, JJ)], semi)

    return carry

  lax.fori_loop(0, nper + 2, step, 0)
  plsc.subcore_barrier()

  def dumpq(q, carry):
    pltpu.async_copy(acc.at[iotabuf.at[q]], stage, semg).wait()
    pltpu.sync_copy(stage, out_hbm.at[c, pl.ds((q0 + q) * 16, 16)])
    return carry

  lax.fori_loop(0, NQ, dumpq, 0)


def _sc_agg(g, src2, dst2, zeros128, iota2, nper):
  return pl.kernel(
      functools.partial(_sc_agg_body, nper),
      out_type=jax.ShapeDtypeStruct((NC, NPAD, HH), jnp.float32),
      mesh=_mesh(),
      scratch_types=[
          pltpu.VMEM((3 * JJ, 16), jnp.int32),
          pltpu.VMEM((3 * JJ, 16), jnp.int32),
          pltpu.VMEM((16, HH), jnp.float32),
          pltpu.VMEM((NQ, 16), jnp.int32),
          pltpu.VMEM((2 * EB, HH), jnp.float32),
          pltpu.VMEM((16, HH), jnp.float32),
          pltpu.VMEM_SHARED((NPAD, HH), jnp.float32),
          pltpu.SemaphoreType.DMA,
          pltpu.SemaphoreType.DMA,
          pltpu.SemaphoreType.DMA,
      ],
  )(g, src2, dst2, zeros128, iota2)


def _dis(deg_ref):
  d = deg_ref[...]
  deg = d[0, :, 0] + d[1, :, 0] + 1.0
  return lax.rsqrt(deg)


def _tc_g0_body(x_ref, w_ref, deg_ref, o_ref):
  dis = _dis(deg_ref)
  o_ref[...] = jnp.dot(x_ref[...], w_ref[...],
                       preferred_element_type=jnp.float32) * dis[:, None]


def _tc_g0(x_pad, W0, degp):
  return pl.pallas_call(
      _tc_g0_body,
      grid=(NB,),
      in_specs=[
          pl.BlockSpec((RTC, HH), lambda i: (i, 0)),
          pl.BlockSpec((HH, HH), lambda i: (0, 0)),
          pl.BlockSpec((NC, RTC, HH), lambda i: (0, i, 0)),
      ],
      out_specs=pl.BlockSpec((RTC, HH), lambda i: (i, 0)),
      out_shape=jax.ShapeDtypeStruct((NPAD, HH), jnp.float32),
  )(x_pad, W0, degp)


def _tc_layer_body(p_ref, g_ref, deg_ref, b_ref, w_ref, o_ref):
  dis = _dis(deg_ref)
  pv = p_ref[...]
  acc = pv[0] + pv[1] + g_ref[...]
  t = jnp.maximum(acc * dis[:, None] + b_ref[...], 0.0)
  o_ref[...] = jnp.dot(t, w_ref[...],
                       preferred_element_type=jnp.float32) * dis[:, None]


def _tc_layer(p, g, degp, b, Wn):
  return pl.pallas_call(
      _tc_layer_body,
      grid=(NB,),
      in_specs=[
          pl.BlockSpec((NC, RTC, HH), lambda i: (0, i, 0)),
          pl.BlockSpec((RTC, HH), lambda i: (i, 0)),
          pl.BlockSpec((NC, RTC, HH), lambda i: (0, i, 0)),
          pl.BlockSpec((1, HH), lambda i: (0, 0)),
          pl.BlockSpec((HH, HH), lambda i: (0, 0)),
      ],
      out_specs=pl.BlockSpec((RTC, HH), lambda i: (i, 0)),
      out_shape=jax.ShapeDtypeStruct((NPAD, HH), jnp.float32),
  )(p, g, degp, b, Wn)


def _tc_final_body(p_ref, g_ref, deg_ref, b_ref, batch_ref, fcw_ref, fcb_ref,
                   f2w_ref, f2b_ref, o_ref, pooled, cnt):
  i = pl.program_id(0)

  @pl.when(i == 0)
  def _():
    pooled[...] = jnp.zeros_like(pooled)
    cnt[...] = jnp.zeros_like(cnt)

  dis = _dis(deg_ref)
  pv = p_ref[...]
  acc = pv[0] + pv[1] + g_ref[...]
  h = jnp.maximum(acc * dis[:, None] + b_ref[...], 0.0)
  bvec = batch_ref[...][0, 0, :]
  onehot = (bvec[:, None] == lax.broadcasted_iota(jnp.int32, (RTC, GG), 1)
            ).astype(jnp.float32)
  pooled[...] += lax.dot_general(onehot, h, (((0,), (0,)), ((), ())),
                                 preferred_element_type=jnp.float32)
  cnt[...] += lax.dot_general(onehot, jnp.ones((RTC, HH), jnp.float32),
                              (((0,), (0,)), ((), ())),
                              preferred_element_type=jnp.float32)

  @pl.when(i == NB - 1)
  def _():
    pm = pooled[...] / jnp.maximum(cnt[...], 1.0)
    z = jnp.maximum(jnp.dot(pm, fcw_ref[...],
                            preferred_element_type=jnp.float32)
                    + fcb_ref[...], 0.0)
    o_ref[...] = jnp.dot(z, f2w_ref[...],
                         preferred_element_type=jnp.float32) + f2b_ref[...]


def _tc_final(p, g, degp, b, batch3, fcW, fcb, f2w, f2b):
  return pl.pallas_call(
      _tc_final_body,
      grid=(NB,),
      in_specs=[
          pl.BlockSpec((NC, RTC, HH), lambda i: (0, i, 0)),
          pl.BlockSpec((RTC, HH), lambda i: (i, 0)),
          pl.BlockSpec((NC, RTC, HH), lambda i: (0, i, 0)),
          pl.BlockSpec((1, HH), lambda i: (0, 0)),
          pl.BlockSpec((1, 1, RTC), lambda i: (i, 0, 0)),
          pl.BlockSpec((HH, HH), lambda i: (0, 0)),
          pl.BlockSpec((1, HH), lambda i: (0, 0)),
          pl.BlockSpec((HH, HH), lambda i: (0, 0)),
          pl.BlockSpec((1, HH), lambda i: (0, 0)),
      ],
      out_specs=pl.BlockSpec((GG, HH), lambda i: (0, 0)),
      out_shape=jax.ShapeDtypeStruct((GG, HH), jnp.float32),
      scratch_shapes=[
          pltpu.VMEM((GG, HH), jnp.float32),
          pltpu.VMEM((GG, HH), jnp.float32),
      ],
  )(p, g, degp, b, batch3, fcW, fcb, f2w, f2b)


def kernel(x, edge_index, batch, W0, b0, W1, b1, W2, b2, fcW, fcb, fc2W, fc2b):
  E = edge_index.shape[1]
  EP = -(-E // (NW * EB)) * (NW * EB)
  nper = EP // (NW * EB)
  src = edge_index[0]
  dst = edge_index[1]
  npadrows = NPAD - NN
  pad_idx = (NN + (jnp.arange(EP - E, dtype=jnp.int32) % npadrows)
             ).astype(jnp.int32)
  src2 = jnp.concatenate([src, pad_idx]).reshape(EP // 16, 16)
  dst2 = jnp.concatenate([dst, pad_idx]).reshape(EP // 16, 16)
  x_pad = jnp.pad(x, ((0, npadrows), (0, 0)))
  zeros128 = jnp.zeros((16, HH), jnp.float32)
  ones128 = jnp.ones((16, HH), jnp.float32)
  iota2 = jnp.arange(NPAD, dtype=jnp.int32).reshape(NPAD // 16, 16)

  degp = _sc_deg(dst2, ones128, zeros128, iota2, nper)
  g0 = _tc_g0(x_pad, W0, degp)
  p = _sc_agg(g0, src2, dst2, zeros128, iota2, nper)
  g1 = _tc_layer(p, g0, degp, b0.reshape(1, HH), W1)
  p = _sc_agg(g1, src2, dst2, zeros128, iota2, nper)
  g2 = _tc_layer(p, g1, degp, b1.reshape(1, HH), W2)
  p = _sc_agg(g2, src2, dst2, zeros128, iota2, nper)

  batch3 = jnp.concatenate(
      [batch.astype(jnp.int32), jnp.full((npadrows,), GG, jnp.int32)]
  ).reshape(NB, 1, RTC)
  f2w = jnp.pad(fc2W, ((0, 0), (0, HH - CC)))
  f2b = jnp.pad(fc2b, (0, HH - CC)).reshape(1, HH)
  out128 = _tc_final(p, g2, degp, b2.reshape(1, HH), batch3,
                     fcW, fcb.reshape(1, HH), f2w, f2b)
  return out128[:, :CC]


# speedup vs baseline: 21.7893x; 1.0019x over previous
import functools
import jax
import jax.numpy as jnp
from jax import lax
from jax.experimental import pallas as pl
from jax.experimental.pallas import tpu as pltpu, tpu_sc as plsc

NN = 10000
HH = 128
GG = 64
CC = 10
NC = 2
NS = 16
NW = NC * NS
EB = 128
NPAD = 10240
RPT = NPAD // NS
RTC = 640
NB = NPAD // RTC

@functools.lru_cache(maxsize=1)
def _mesh():
  return plsc.VectorSubcoreMesh(
      core_axis_name="c", subcore_axis_name="s", num_cores=NC, num_subcores=NS)


JJ = EB // 16
NQ = RPT // 16


def _sc_deg_body(nper, dst_hbm, ones_hbm, zeros_hbm, iota_hbm, out_hbm,
                 idxd, onesbuf, zbuf, iotabuf, stage, acc, sem, semi):
  c = lax.axis_index("c")
  s = lax.axis_index("s")
  wid = s * NC + c
  q0 = s * NQ
  pltpu.sync_copy(ones_hbm, onesbuf)
  pltpu.sync_copy(zeros_hbm, zbuf)
  pltpu.sync_copy(iota_hbm.at[pl.ds(q0, NQ)], iotabuf)

  def initq(q, carry):
    pltpu.sync_copy(zbuf, acc.at[iotabuf.at[q]])
    return carry

  lax.fori_loop(0, NQ, initq, 0)
  plsc.subcore_barrier()

  ebase = wid * nper * JJ
  pltpu.async_copy(dst_hbm.at[pl.ds(ebase, JJ)], idxd.at[pl.ds(0, JJ)], semi)

  def chunk(i, carry):
    k0 = lax.rem(i, 3) * JJ
    k1 = lax.rem(i + 1, 3) * JJ

    @pl.when(i < nper)
    def _():
      pltpu.make_async_copy(dst_hbm.at[pl.ds(ebase, JJ)],
                            idxd.at[pl.ds(k0, JJ)], semi).wait()
      for j in range(JJ):
        pltpu.async_copy(onesbuf, acc.at[idxd.at[k0 + j]], sem, add=True)

    @pl.when(i + 1 < nper)
    def _():
      pltpu.async_copy(dst_hbm.at[pl.ds(ebase + (i + 1) * JJ, JJ)],
                       idxd.at[pl.ds(k1, JJ)], semi)

    @pl.when(i >= 1)
    def _():
      for j in range(JJ):
        pltpu.make_async_copy(onesbuf, acc.at[iotabuf.at[0]], sem).wait()

    return carry

  lax.fori_loop(0, nper + 1, chunk, 0)
  plsc.subcore_barrier()

  def dumpq(q, carry):
    pltpu.async_copy(acc.at[iotabuf.at[q]], stage, sem).wait()
    pltpu.sync_copy(stage, out_hbm.at[c, pl.ds((q0 + q) * 16, 16)])
    return carry

  lax.fori_loop(0, NQ, dumpq, 0)


def _sc_deg(dst2, ones128, zeros128, iota2, nper):
  return pl.kernel(
      functools.partial(_sc_deg_body, nper),
      out_type=jax.ShapeDtypeStruct((NC, NPAD, HH), jnp.float32),
      mesh=_mesh(),
      scratch_types=[
          pltpu.VMEM((3 * JJ, 16), jnp.int32),
          pltpu.VMEM((16, HH), jnp.float32),
          pltpu.VMEM((16, HH), jnp.float32),
          pltpu.VMEM((NQ, 16), jnp.int32),
          pltpu.VMEM((16, HH), jnp.float32),
          pltpu.VMEM_SHARED((NPAD, HH), jnp.float32),
          pltpu.SemaphoreType.DMA,
          pltpu.SemaphoreType.DMA,
      ],
  )(dst2, ones128, zeros128, iota2)


def _sc_agg_body(nper, g_hbm, src_hbm, dst_hbm, zeros_hbm, iota_hbm, out_hbm,
                 idxs3, idxd3, zbuf, iotabuf, rowbuf2, stage, acc,
                 semg, sems, semi):
  c = lax.axis_index("c")
  s = lax.axis_index("s")
  wid = s * NC + c
  q0 = s * NQ
  pltpu.sync_copy(zeros_hbm, zbuf)
  pltpu.sync_copy(iota_hbm.at[pl.ds(q0, NQ)], iotabuf)
  ebase = wid * nper * JJ

  def initq(q, carry):
    pltpu.sync_copy(zbuf, acc.at[iotabuf.at[q]])
    return carry

  lax.fori_loop(0, NQ, initq, 0)
  pltpu.async_copy(src_hbm.at[pl.ds(ebase, JJ)], idxs3.at[pl.ds(0, JJ)], semi)
  pltpu.async_copy(dst_hbm.at[pl.ds(ebase, JJ)], idxd3.at[pl.ds(0, JJ)], semi)
  plsc.subcore_barrier()

  def step(i, carry):
    par = lax.rem(i, 2) * EB
    ppar = lax.rem(i + 1, 2) * EB
    k0 = lax.rem(i, 3) * JJ
    k1 = lax.rem(i + 1, 3) * JJ
    k2 = lax.rem(i + 2, 3) * JJ

    @pl.when(i >= 2)
    def _():
      for j in range(JJ):
        pltpu.make_async_copy(rowbuf2.at[pl.ds(par + 16 * j, 16)],
                              acc.at[iotabuf.at[0]], sems).wait()

    @pl.when(i < nper)
    def _():
      pltpu.make_async_copy(src_hbm.at[pl.ds(ebase, JJ)],
                            idxs3.at[pl.ds(k0, JJ)], semi).wait()
      pltpu.make_async_copy(dst_hbm.at[pl.ds(ebase, JJ)],
                            idxd3.at[pl.ds(k0, JJ)], semi).wait()
      for j in range(JJ):
        pltpu.async_copy(g_hbm.at[idxs3.at[k0 + j]],
                         rowbuf2.at[pl.ds(par + 16 * j, 16)], semg)

    @pl.when(jnp.logical_and(i >= 1, i <= nper))
    def _():
      for j in range(JJ):
        pltpu.make_async_copy(g_hbm.at[iotabuf.at[0]],
                              rowbuf2.at[pl.ds(ppar + 16 * j, 16)],
                              semg).wait()
      for j in range(JJ):
        pltpu.async_copy(rowbuf2.at[pl.ds(ppar + 16 * j, 16)],
                         acc.at[idxd3.at[k2 + j]], sems, add=True)

    @pl.when(i + 1 < nper)
    def _():
      nbase = ebase + (i + 1) * JJ
      pltpu.async_copy(src_hbm.at[pl.ds(nbase, JJ)],
                       idxs3.at[pl.ds(k1, JJ)], semi)
      pltpu.async_copy(dst_hbm.at[pl.ds(nbase, JJ)],
                       idxd3.at[pl.ds(k1, JJ)], semi)

    return carry

  lax.fori_loop(0, nper + 2, step, 0)
  plsc.subcore_barrier()

  def dumpq(q, carry):
    pltpu.async_copy(acc.at[iotabuf.at[q]], stage, semg).wait()
    pltpu.sync_copy(stage, out_hbm.at[c, pl.ds((q0 + q) * 16, 16)])
    return carry

  lax.fori_loop(0, NQ, dumpq, 0)


def _sc_agg(g, src2, dst2, zeros128, iota2, nper):
  return pl.kernel(
      functools.partial(_sc_agg_body, nper),
      out_type=jax.ShapeDtypeStruct((NC, NPAD, HH), jnp.float32),
      mesh=_mesh(),
      scratch_types=[
          pltpu.VMEM((3 * JJ, 16), jnp.int32),
          pltpu.VMEM((3 * JJ, 16), jnp.int32),
          pltpu.VMEM((16, HH), jnp.float32),
          pltpu.VMEM((NQ, 16), jnp.int32),
          pltpu.VMEM((2 * EB, HH), jnp.float32),
          pltpu.VMEM((16, HH), jnp.float32),
          pltpu.VMEM_SHARED((NPAD, HH), jnp.float32),
          pltpu.SemaphoreType.DMA,
          pltpu.SemaphoreType.DMA,
          pltpu.SemaphoreType.DMA,
      ],
  )(g, src2, dst2, zeros128, iota2)


def _dis(deg_ref):
  d = deg_ref[...]
  deg = d[0, :, 0] + d[1, :, 0] + 1.0
  return lax.rsqrt(deg)


def _tc_g0_body(x_ref, w_ref, deg_ref, o_ref, d_ref):
  dis = _dis(deg_ref)
  d_ref[...] = dis[None, None, :]
  o_ref[...] = jnp.dot(x_ref[...], w_ref[...],
                       preferred_element_type=jnp.float32) * dis[:, None]


def _tc_g0(x_pad, W0, degp):
  return pl.pallas_call(
      _tc_g0_body,
      grid=(NB,),
      in_specs=[
          pl.BlockSpec((RTC, HH), lambda i: (i, 0)),
          pl.BlockSpec((HH, HH), lambda i: (0, 0)),
          pl.BlockSpec((NC, RTC, HH), lambda i: (0, i, 0)),
      ],
      out_specs=[
          pl.BlockSpec((RTC, HH), lambda i: (i, 0)),
          pl.BlockSpec((1, 1, RTC), lambda i: (i, 0, 0)),
      ],
      out_shape=[
          jax.ShapeDtypeStruct((NPAD, HH), jnp.float32),
          jax.ShapeDtypeStruct((NB, 1, RTC), jnp.float32),
      ],
  )(x_pad, W0, degp)


def _tc_layer_body(p_ref, g_ref, d_ref, b_ref, w_ref, o_ref):
  dis = d_ref[...][0, 0, :]
  pv = p_ref[...]
  acc = pv[0] + pv[1] + g_ref[...]
  t = jnp.maximum(acc * dis[:, None] + b_ref[...], 0.0)
  o_ref[...] = jnp.dot(t, w_ref[...],
                       preferred_element_type=jnp.float32) * dis[:, None]


def _tc_layer(p, g, dis3, b, Wn):
  return pl.pallas_call(
      _tc_layer_body,
      grid=(NB,),
      in_specs=[
          pl.BlockSpec((NC, RTC, HH), lambda i: (0, i, 0)),
          pl.BlockSpec((RTC, HH), lambda i: (i, 0)),
          pl.BlockSpec((1, 1, RTC), lambda i: (i, 0, 0)),
          pl.BlockSpec((1, HH), lambda i: (0, 0)),
          pl.BlockSpec((HH, HH), lambda i: (0, 0)),
      ],
      out_specs=pl.BlockSpec((RTC, HH), lambda i: (i, 0)),
      out_shape=jax.ShapeDtypeStruct((NPAD, HH), jnp.float32),
  )(p, g, dis3, b, Wn)


def _tc_final_body(p_ref, g_ref, d_ref, b_ref, batch_ref, fcw_ref, fcb_ref,
                   f2w_ref, f2b_ref, o_ref, pooled, cnt):
  i = pl.program_id(0)

  @pl.when(i == 0)
  def _():
    pooled[...] = jnp.zeros_like(pooled)
    cnt[...] = jnp.zeros_like(cnt)

  dis = d_ref[...][0, 0, :]
  pv = p_ref[...]
  acc = pv[0] + pv[1] + g_ref[...]
  h = jnp.maximum(acc * dis[:, None] + b_ref[...], 0.0)
  bvec = batch_ref[...][0, 0, :]
  onehot = (bvec[:, None] == lax.broadcasted_iota(jnp.int32, (RTC, GG), 1)
            ).astype(jnp.float32)
  pooled[...] += lax.dot_general(onehot, h, (((0,), (0,)), ((), ())),
                                 preferred_element_type=jnp.float32)
  cnt[...] += lax.dot_general(onehot, jnp.ones((RTC, HH), jnp.float32),
                              (((0,), (0,)), ((), ())),
                              preferred_element_type=jnp.float32)

  @pl.when(i == NB - 1)
  def _():
    pm = pooled[...] / jnp.maximum(cnt[...], 1.0)
    z = jnp.maximum(jnp.dot(pm, fcw_ref[...],
                            preferred_element_type=jnp.float32)
                    + fcb_ref[...], 0.0)
    o_ref[...] = jnp.dot(z, f2w_ref[...],
                         preferred_element_type=jnp.float32) + f2b_ref[...]


def _tc_final(p, g, dis3, b, batch3, fcW, fcb, f2w, f2b):
  return pl.pallas_call(
      _tc_final_body,
      grid=(NB,),
      in_specs=[
          pl.BlockSpec((NC, RTC, HH), lambda i: (0, i, 0)),
          pl.BlockSpec((RTC, HH), lambda i: (i, 0)),
          pl.BlockSpec((1, 1, RTC), lambda i: (i, 0, 0)),
          pl.BlockSpec((1, HH), lambda i: (0, 0)),
          pl.BlockSpec((1, 1, RTC), lambda i: (i, 0, 0)),
          pl.BlockSpec((HH, HH), lambda i: (0, 0)),
          pl.BlockSpec((1, HH), lambda i: (0, 0)),
          pl.BlockSpec((HH, HH), lambda i: (0, 0)),
          pl.BlockSpec((1, HH), lambda i: (0, 0)),
      ],
      out_specs=pl.BlockSpec((GG, HH), lambda i: (0, 0)),
      out_shape=jax.ShapeDtypeStruct((GG, HH), jnp.float32),
      scratch_shapes=[
          pltpu.VMEM((GG, HH), jnp.float32),
          pltpu.VMEM((GG, HH), jnp.float32),
      ],
  )(p, g, dis3, b, batch3, fcW, fcb, f2w, f2b)


def kernel(x, edge_index, batch, W0, b0, W1, b1, W2, b2, fcW, fcb, fc2W, fc2b):
  E = edge_index.shape[1]
  EP = -(-E // (NW * EB)) * (NW * EB)
  nper = EP // (NW * EB)
  src = edge_index[0]
  dst = edge_index[1]
  npadrows = NPAD - NN
  pad_idx = (NN + (jnp.arange(EP - E, dtype=jnp.int32) % npadrows)
             ).astype(jnp.int32)
  src2 = jnp.concatenate([src, pad_idx]).reshape(EP // 16, 16)
  dst2 = jnp.concatenate([dst, pad_idx]).reshape(EP // 16, 16)
  x_pad = jnp.pad(x, ((0, npadrows), (0, 0)))
  zeros128 = jnp.zeros((16, HH), jnp.float32)
  ones128 = jnp.ones((16, HH), jnp.float32)
  iota2 = jnp.arange(NPAD, dtype=jnp.int32).reshape(NPAD // 16, 16)

  degp = _sc_deg(dst2, ones128, zeros128, iota2, nper)
  g0, dis3 = _tc_g0(x_pad, W0, degp)
  p = _sc_agg(g0, src2, dst2, zeros128, iota2, nper)
  g1 = _tc_layer(p, g0, dis3, b0.reshape(1, HH), W1)
  p = _sc_agg(g1, src2, dst2, zeros128, iota2, nper)
  g2 = _tc_layer(p, g1, dis3, b1.reshape(1, HH), W2)
  p = _sc_agg(g2, src2, dst2, zeros128, iota2, nper)

  batch3 = jnp.concatenate(
      [batch.astype(jnp.int32), jnp.full((npadrows,), GG, jnp.int32)]
  ).reshape(NB, 1, RTC)
  f2w = jnp.pad(fc2W, ((0, 0), (0, HH - CC)))
  f2b = jnp.pad(fc2b, (0, HH - CC)).reshape(1, HH)
  out128 = _tc_final(p, g2, dis3, b2.reshape(1, HH), batch3,
                     fcW, fcb.reshape(1, HH), f2w, f2b)
  return out128[:, :CC]
